# baseline (speedup 1.0000x reference)
import functools

import jax
import jax.numpy as jnp
from jax import lax
from jax.experimental import pallas as pl
from jax.experimental.pallas import tpu as pltpu
from jax.experimental.pallas import tpu_sc as plsc

B = 16384
CONT = 13
NCAT = 26
VOCAB = 100000
D = 64

NC = 2
NS = 16
NW = NC * NS
ROWS_W = B // NW
NB = 32
NCHUNK = ROWS_W // NB
GIDX = NB * NCAT
GG = 64
NGG = GIDX // GG
CIDX = NB * CONT
CG = 32
NCG = CIDX // CG


def _sc_body(xcat_hbm, xcont_hbm, tab_hbm, ctab_hbm, goffs_hbm, soffs_hbm,
             coffs_hbm, out_hbm,
             gidx_v, sidx_v, csidx_v, rows_v, xcont_v, cont_v,
             goffs_v, soffs_v, coffs_v, ctab_v,
             gsem, ssem, csem):
  w = lax.axis_index("s") * NC + lax.axis_index("c")
  base = w * ROWS_W

  pltpu.sync_copy(ctab_hbm, ctab_v)
  pltpu.sync_copy(goffs_hbm, goffs_v)
  pltpu.sync_copy(soffs_hbm, soffs_v)
  pltpu.sync_copy(coffs_hbm, coffs_v)

  def chunk(ch, carry):
    r0 = base + ch * NB

    pltpu.sync_copy(xcat_hbm.at[pl.ds(r0 * NCAT, GIDX)], gidx_v)
    for i in range(GIDX // 16):
      sl = pl.ds(i * 16, 16)
      gidx_v[sl] = gidx_v[sl] + goffs_v[sl]

    gds = []
    for g in range(NGG):
      gds.append(pltpu.async_copy(
          tab_hbm.at[gidx_v.at[pl.ds(g * GG, GG)]],
          rows_v.at[pl.ds(g * GG, GG)], gsem))

    o0 = r0 * (CONT + NCAT)
    for g in range(NGG):
      for j in range(GG // 16):
        sl = pl.ds(j * 16, 16)
        sidx_v[g, sl] = soffs_v[g, sl] + o0
    for g in range(NCG):
      for j in range(CG // 16):
        sl = pl.ds(j * 16, 16)
        csidx_v[g, sl] = coffs_v[g, sl] + o0

    pltpu.sync_copy(xcont_hbm.at[pl.ds(r0 * CONT, CIDX)],
                    xcont_v.at[pl.ds(0, CIDX)])

    def cont_row(b, c2):
      v = xcont_v[pl.ds(b * CONT, 16)]
      for c in range(CONT):
        s = v[c]
        for d in range(D // 16):
          sl = pl.ds(d * 16, 16)
          cont_v[b * CONT + c, sl] = s * ctab_v[c, sl]
      return c2
    lax.fori_loop(0, NB, cont_row, 0)

    cds = []
    for g in range(NCG):
      cds.append(pltpu.async_copy(
          cont_v.at[pl.ds(g * CG, CG)],
          out_hbm.at[csidx_v.at[g]], csem))

    for dsc in gds:
      dsc.wait()
    sds = []
    for g in range(NGG):
      sds.append(pltpu.async_copy(
          rows_v.at[pl.ds(g * GG, GG)],
          out_hbm.at[sidx_v.at[g]], ssem))

    for dsc in cds:
      dsc.wait()
    for dsc in sds:
      dsc.wait()
    return carry

  lax.fori_loop(0, NCHUNK, chunk, 0)


@jax.jit
def kernel(x_cont, x_cat, cat_tables, cont_table):
  f32 = jnp.float32
  i32 = jnp.int32
  xcat_flat = x_cat.astype(i32).reshape(B * NCAT)
  xcont_flat = x_cont.astype(f32).reshape(B * CONT)
  tab_flat = cat_tables.reshape(NCAT * VOCAB, D)

  goffs = jnp.tile(jnp.arange(NCAT, dtype=i32) * VOCAB, NB)
  bb = jnp.arange(NB, dtype=i32)[:, None]
  soffs = (bb * (CONT + NCAT) + CONT
           + jnp.arange(NCAT, dtype=i32)[None, :]).reshape(NGG, GG)
  coffs = (bb * (CONT + NCAT)
           + jnp.arange(CONT, dtype=i32)[None, :]).reshape(NCG, CG)

  mesh = plsc.VectorSubcoreMesh(core_axis_name="c", subcore_axis_name="s",
                                num_cores=NC, num_subcores=NS)
  out_flat = pl.kernel(
      _sc_body,
      out_type=jax.ShapeDtypeStruct((B * (CONT + NCAT), D), f32),
      mesh=mesh,
      compiler_params=pltpu.CompilerParams(use_tc_tiling_on_sc=False),
      scratch_types=[
          pltpu.VMEM((GIDX,), i32),
          pltpu.VMEM((NGG, GG), i32),
          pltpu.VMEM((NCG, CG), i32),
          pltpu.VMEM((GIDX, D), f32),
          pltpu.VMEM((CIDX + 16,), f32),
          pltpu.VMEM((CIDX, D), f32),
          pltpu.VMEM((GIDX,), i32),
          pltpu.VMEM((NGG, GG), i32),
          pltpu.VMEM((NCG, CG), i32),
          pltpu.VMEM((CONT, D), f32),
          pltpu.SemaphoreType.DMA,
          pltpu.SemaphoreType.DMA,
          pltpu.SemaphoreType.DMA,
      ],
  )(xcat_flat, xcont_flat, tab_flat, cont_table, goffs, soffs, coffs)
  return out_flat.reshape(B, CONT + NCAT, D)


# speedup vs baseline: 5.4841x; 5.4841x over previous
import functools

import jax
import jax.numpy as jnp
from jax import lax
from jax.experimental import pallas as pl
from jax.experimental.pallas import tpu as pltpu
from jax.experimental.pallas import tpu_sc as plsc

B = 16384
CONT = 13
NCAT = 26
VOCAB = 100000
D = 64

NC = 2
NS = 16
NW = NC * NS
CAT_ROWS = NCAT * D
CONT_ROWS = CONT * D
CAT_PW = CAT_ROWS // NW
CONT_PW = CONT_ROWS // NW
CHUNK = 1024
NCHUNK = B // CHUNK
NSLOT = 8
CQ = 4 * CHUNK


def _sc_body(xcat_hbm, xcont_hbm, tab_hbm, ctab_hbm, out_hbm,
             row_v, col_v, colq_v, ring_v, ctab_v, rsem, ssem):
  w = lax.axis_index("s") * NC + lax.axis_index("c")

  def drain_stores():
    pltpu.make_async_copy(
        tab_hbm.at[0, 0, pl.ds(0, NSLOT * CHUNK)], ring_v, ssem).wait()

  u0 = w * CAT_PW
  pltpu.async_copy(tab_hbm.at[u0 // D, u0 % D], row_v, rsem)

  def item(k, carry):
    f_prev, c_prev = carry
    u = w * CAT_PW + k
    f = u // D
    d = u % D

    @pl.when(k > 0)
    def _():
      drain_stores()
    @pl.when(f != f_prev)
    def _():
      pltpu.sync_copy(xcat_hbm.at[f], col_v)

    v = w * CONT_PW + k // 2
    cc = v // D
    dc = v % D
    h = k % 2
    @pl.when(cc != c_prev)
    def _():
      pltpu.sync_copy(ctab_hbm.at[cc], ctab_v)
    s_vec = plsc.load_gather(ctab_v, [jnp.full((16,), dc, dtype=jnp.int32)])

    sds = []
    for i in range(NCHUNK // 2):
      if i % 4 == 0:
        pltpu.sync_copy(
            xcont_hbm.at[cc, pl.ds((h * 2 + i // 4) * CQ, CQ)], colq_v)
      c_abs = h * (NCHUNK // 2) + i

      @plsc.parallel_loop(0, CHUNK, step=16, unroll=8)
      def _(off):
        xv = plsc.bitcast(colq_v[pl.ds((i % 4) * CHUNK + off, 16)],
                          jnp.float32)
        ring_v[pl.ds(i * CHUNK + off, 16)] = xv * s_vec

      sds.append(pltpu.async_copy(
          ring_v.at[pl.ds(i * CHUNK, CHUNK)],
          out_hbm.at[cc, dc, pl.ds(c_abs * CHUNK, CHUNK)], ssem))

    pltpu.make_async_copy(tab_hbm.at[f, d], row_v, rsem).wait()

    for c in range(NCHUNK):
      slot = c % NSLOT
      sds[c].wait()

      @plsc.parallel_loop(0, CHUNK, step=16, unroll=8)
      def _(off):
        idx = col_v[pl.ds(c * CHUNK + off, 16)]
        ring_v[pl.ds(slot * CHUNK + off, 16)] = plsc.load_gather(
            row_v, [idx])

      sds.append(pltpu.async_copy(
          ring_v.at[pl.ds(slot * CHUNK, CHUNK)],
          out_hbm.at[CONT + f, d, pl.ds(c * CHUNK, CHUNK)], ssem))

    @pl.when(k < CAT_PW - 1)
    def _():
      u2 = u + 1
      pltpu.async_copy(tab_hbm.at[u2 // D, u2 % D], row_v, rsem)

    return f, cc

  lax.fori_loop(0, CAT_PW, item, (jnp.int32(-1), jnp.int32(-1)))
  drain_stores()


@jax.jit
def kernel(x_cont, x_cat, cat_tables, cont_table):
  f32 = jnp.float32
  i32 = jnp.int32
  tabT = jnp.transpose(cat_tables, (0, 2, 1))
  xcatT = x_cat.astype(i32).T
  xcontT = lax.bitcast_convert_type(x_cont.T, i32)

  mesh = plsc.VectorSubcoreMesh(core_axis_name="c", subcore_axis_name="s",
                                num_cores=NC, num_subcores=NS)
  out = pl.kernel(
      _sc_body,
      out_type=jax.ShapeDtypeStruct((CONT + NCAT, D, B), f32),
      mesh=mesh,
      compiler_params=pltpu.CompilerParams(
          use_tc_tiling_on_sc=True, needs_layout_passes=False),
      scratch_types=[
          pltpu.VMEM((VOCAB,), f32),
          pltpu.VMEM((B,), i32),
          pltpu.VMEM((CQ,), i32),
          pltpu.VMEM((NSLOT * CHUNK,), f32),
          pltpu.VMEM((D,), f32),
          pltpu.SemaphoreType.DMA,
          pltpu.SemaphoreType.DMA,
      ],
  )(xcatT, xcontT, tabT, cont_table)
  return jnp.transpose(out, (2, 0, 1))
